# baseline (speedup 1.0000x reference)
import functools
import numpy as np
import jax
import jax.numpy as jnp
from jax import lax
from jax.experimental import pallas as pl
from jax.experimental.pallas import tpu as pltpu
from jax.experimental.pallas import tpu_sc as plsc

_EPS = 1e-5
_MAXLEN = 512

_NW = 32
_C = 640
_KBUF = 2
_G = 4


def _pe_table(S, D):
    pos = np.arange(_MAXLEN, dtype=np.float32)[:, None]
    i = np.arange(0, D, 2, dtype=np.float32)
    div = np.exp(-(np.log(10000.0)) * i / D)
    pe = np.zeros((_MAXLEN, D), dtype=np.float32)
    pe[:, 0::2] = np.sin(pos * div)
    pe[:, 1::2] = np.cos(pos * div)
    return jnp.asarray(pe[:S])


def _make_sc_gather(n_pairs_w, n_chunks, E):
    mesh = plsc.VectorSubcoreMesh(core_axis_name="c", subcore_axis_name="s")
    idx_scratch = [pltpu.VMEM((_C,), jnp.int32) for _ in range(_KBUF)]
    row_scratch = [pltpu.VMEM((_C, E), jnp.float32) for _ in range(_KBUF)]
    HC = _C // 2

    @functools.partial(
        pl.kernel,
        mesh=mesh,
        out_type=jax.ShapeDtypeStruct((_NW * n_pairs_w, 2 * E), jnp.float32),
        scratch_types=idx_scratch + row_scratch + [pltpu.SemaphoreType.DMA],
        compiler_params=pltpu.CompilerParams(use_tc_tiling_on_sc=False),
    )
    def sc_gather(seq_hbm, table_hbm, out_hbm, *scratch):
        idx_v = scratch[:_KBUF]
        rows_v = scratch[_KBUF : 2 * _KBUF]
        sem = scratch[2 * _KBUF]
        wid = lax.axis_index("s") * 2 + lax.axis_index("c")

        def group(g, _):
            copies = []
            for bf in range(_KBUF):
                pltpu.sync_copy(seq_hbm.at[wid, g * _KBUF + bf], idx_v[bf])
                cp = pltpu.async_copy(
                    table_hbm.at[idx_v[bf]], rows_v[bf], sem
                )
                copies.append(cp)
            for bf in range(_KBUF):
                pairbase = wid * n_pairs_w + (g * _KBUF + bf) * HC
                copies[bf].wait()
                pltpu.sync_copy(
                    rows_v[bf].at[pl.ds(0, HC)],
                    out_hbm.at[pl.ds(pairbase, HC), pl.ds(0, E)],
                )
                pltpu.sync_copy(
                    rows_v[bf].at[pl.ds(HC, HC)],
                    out_hbm.at[pl.ds(pairbase, HC), pl.ds(E, E)],
                )
            return _

        lax.fori_loop(0, n_chunks // _KBUF, group, None)

    return sc_gather


def _tc_body(BB, HS, seq_ref, emb_ref, wlo_ref, whi_ref, pb_ref, g_ref,
             bt_ref, out_ref):
    wlo = wlo_ref[...]
    whi = whi_ref[...]
    pb = pb_ref[...]
    gm = g_ref[...]
    bt = bt_ref[...]
    flags = jnp.transpose(
        (seq_ref[...] != 0).astype(jnp.float32)
    )

    def ln(x):
        mu = jnp.mean(x, axis=-1, keepdims=True)
        xc = x - mu
        var = jnp.mean(xc * xc, axis=-1, keepdims=True)
        return (xc * lax.rsqrt(var + _EPS)) * gm + bt

    p_all = emb_ref[...]
    x_lo_all = jnp.dot(p_all, wlo, preferred_element_type=jnp.float32)
    x_hi_all = jnp.dot(p_all, whi, preferred_element_type=jnp.float32)
    for i in range(BB):
        m_lo = flags[0:HS, i : i + 1]
        m_hi = flags[HS : 2 * HS, i : i + 1]
        x_lo = x_lo_all[i * HS : (i + 1) * HS]
        out_ref[i, 0:HS] = ln(x_lo * m_lo + pb[0:HS])
        x_hi = x_hi_all[i * HS : (i + 1) * HS]
        out_ref[i, HS : 2 * HS] = ln(x_hi * m_hi + pb[HS : 2 * HS])


def _tc_body_chunk(BB, HS, prev_unused, *rest):
    del prev_unused
    _tc_body(BB, HS, *rest)


def kernel(sequence, table, W, b, gamma, beta):
    B, S = sequence.shape
    V, E = table.shape
    D = W.shape[0]
    N = B * S
    HS = S // 2
    Ng = N // _G
    n_w = Ng // _NW
    n_pairs_w = n_w // 2
    assert S % 2 == 0 and n_w % (_KBUF * _C) == 0 and B % _G == 0
    n_chunks = n_w // _C

    seq_i32 = sequence.astype(jnp.int32)
    lo = seq_i32[:, :HS].reshape(_G, _NW, n_chunks, _C // 2)
    hi = seq_i32[:, HS:].reshape(_G, _NW, n_chunks, _C // 2)
    seq_flat = jnp.concatenate([lo, hi], axis=-1)

    sc_gather = _make_sc_gather(n_pairs_w, n_chunks, E)
    pb = _pe_table(S, D) + b[None, :]
    wlo = jnp.concatenate([W.T, jnp.zeros((E, D), jnp.float32)], axis=0)
    whi = jnp.concatenate([jnp.zeros((E, D), jnp.float32), W.T], axis=0)
    gm = gamma.reshape(1, D)
    bt = beta.reshape(1, D)

    BB = 16
    Bg = B // _G
    grid = (Bg // BB,)
    out_shape = jax.ShapeDtypeStruct((B, S, D), jnp.float32)

    embs = [sc_gather(seq_flat[g], table) for g in range(_G)]

    out = None
    for g in range(_G):
        base = g * (Bg // BB)

        def shifted(b0, i):
            return (b0 + i, 0, 0)

        def shifted2(b0, i):
            return (b0 + i, 0)

        seq_spec = pl.BlockSpec(
            (BB, S), functools.partial(shifted2, base)
        )
        emb_spec = pl.BlockSpec((BB * HS, 2 * E), lambda i: (i, 0))
        common_specs = [
            pl.BlockSpec((2 * E, D), lambda i: (0, 0)),
            pl.BlockSpec((2 * E, D), lambda i: (0, 0)),
            pl.BlockSpec((S, D), lambda i: (0, 0)),
            pl.BlockSpec((1, D), lambda i: (0, 0)),
            pl.BlockSpec((1, D), lambda i: (0, 0)),
        ]
        out_spec = pl.BlockSpec((BB, S, D), functools.partial(shifted, base))
        cp = pltpu.CompilerParams(dimension_semantics=("arbitrary",))
        if g == 0:
            out = pl.pallas_call(
                functools.partial(_tc_body, BB, HS),
                grid=grid,
                in_specs=[seq_spec, emb_spec] + common_specs,
                out_specs=out_spec,
                out_shape=out_shape,
                compiler_params=cp,
            )(seq_i32, embs[g], wlo, whi, pb, gm, bt)
        else:
            out = pl.pallas_call(
                functools.partial(_tc_body_chunk, BB, HS),
                grid=grid,
                in_specs=[pl.BlockSpec(memory_space=pl.ANY), seq_spec,
                          emb_spec] + common_specs,
                out_specs=out_spec,
                out_shape=out_shape,
                input_output_aliases={0: 0},
                compiler_params=cp,
            )(out, seq_i32, embs[g], wlo, whi, pb, gm, bt)
    return out


# speedup vs baseline: 1.2447x; 1.2447x over previous
import functools
import numpy as np
import jax
import jax.numpy as jnp
from jax import lax
from jax.experimental import pallas as pl
from jax.experimental.pallas import tpu as pltpu
from jax.experimental.pallas import tpu_sc as plsc

_EPS = 1e-5
_MAXLEN = 512

_NW = 32
_C = 640
_KBUF = 2
_G = 4


def _pe_table(S, D):
    pos = np.arange(_MAXLEN, dtype=np.float32)[:, None]
    i = np.arange(0, D, 2, dtype=np.float32)
    div = np.exp(-(np.log(10000.0)) * i / D)
    pe = np.zeros((_MAXLEN, D), dtype=np.float32)
    pe[:, 0::2] = np.sin(pos * div)
    pe[:, 1::2] = np.cos(pos * div)
    return jnp.asarray(pe[:S])


def _make_sc_gather(n_pairs_w, n_chunks, E):
    mesh = plsc.VectorSubcoreMesh(core_axis_name="c", subcore_axis_name="s")
    idx_scratch = [pltpu.VMEM((_C,), jnp.int32) for _ in range(_KBUF)]
    row_scratch = [pltpu.VMEM((_C, E), jnp.float32) for _ in range(_KBUF)]
    HC = _C // 2

    @functools.partial(
        pl.kernel,
        mesh=mesh,
        out_type=jax.ShapeDtypeStruct((_NW * n_pairs_w, 2 * E), jnp.float32),
        scratch_types=idx_scratch + row_scratch + [pltpu.SemaphoreType.DMA],
        compiler_params=pltpu.CompilerParams(use_tc_tiling_on_sc=False),
    )
    def sc_gather(seq_hbm, table_hbm, out_hbm, *scratch):
        idx_v = scratch[:_KBUF]
        rows_v = scratch[_KBUF : 2 * _KBUF]
        sem = scratch[2 * _KBUF]
        wid = lax.axis_index("s") * 2 + lax.axis_index("c")

        def group(g, _):
            copies = []
            for bf in range(_KBUF):
                pltpu.sync_copy(seq_hbm.at[wid, g * _KBUF + bf], idx_v[bf])
                cp = pltpu.async_copy(
                    table_hbm.at[idx_v[bf]], rows_v[bf], sem
                )
                copies.append(cp)
            for bf in range(_KBUF):
                pairbase = wid * n_pairs_w + (g * _KBUF + bf) * HC
                copies[bf].wait()
                pltpu.sync_copy(
                    rows_v[bf].at[pl.ds(0, HC)],
                    out_hbm.at[pl.ds(pairbase, HC), pl.ds(0, E)],
                )
                pltpu.sync_copy(
                    rows_v[bf].at[pl.ds(HC, HC)],
                    out_hbm.at[pl.ds(pairbase, HC), pl.ds(E, E)],
                )
            return _

        lax.fori_loop(0, n_chunks // _KBUF, group, None)

    return sc_gather


def _tc_body(BB, HS, emb_ref, wlo_ref, whi_ref, pb_ref, g_ref,
             bt_ref, out_ref):
    wlo = wlo_ref[...]
    whi = whi_ref[...]
    pb = pb_ref[...]
    gm = g_ref[...]
    bt = bt_ref[...]

    def ln(x):
        mu = jnp.mean(x, axis=-1, keepdims=True)
        xc = x - mu
        var = jnp.mean(xc * xc, axis=-1, keepdims=True)
        return (xc * lax.rsqrt(var + _EPS)) * gm + bt

    p_all = emb_ref[...]
    x_lo_all = jnp.dot(p_all, wlo, preferred_element_type=jnp.float32)
    x_hi_all = jnp.dot(p_all, whi, preferred_element_type=jnp.float32)
    for i in range(BB):
        x_lo = x_lo_all[i * HS : (i + 1) * HS]
        out_ref[i, 0:HS] = ln(x_lo + pb[0:HS])
        x_hi = x_hi_all[i * HS : (i + 1) * HS]
        out_ref[i, HS : 2 * HS] = ln(x_hi + pb[HS : 2 * HS])


def _tc_body_chunk(BB, HS, prev_unused, *rest):
    del prev_unused
    _tc_body(BB, HS, *rest)


def kernel(sequence, table, W, b, gamma, beta):
    B, S = sequence.shape
    V, E = table.shape
    D = W.shape[0]
    N = B * S
    HS = S // 2
    Ng = N // _G
    n_w = Ng // _NW
    n_pairs_w = n_w // 2
    assert S % 2 == 0 and n_w % (_KBUF * _C) == 0 and B % _G == 0
    n_chunks = n_w // _C

    table_ext = jnp.concatenate(
        [table, jnp.zeros((8, E), jnp.float32)], axis=0
    )
    seq_i32 = sequence.astype(jnp.int32)
    seq_remap = jnp.where(seq_i32 == 0, V, seq_i32)
    lo = seq_remap[:, :HS].reshape(_G, _NW, n_chunks, _C // 2)
    hi = seq_remap[:, HS:].reshape(_G, _NW, n_chunks, _C // 2)
    seq_flat = jnp.concatenate([lo, hi], axis=-1)

    sc_gather = _make_sc_gather(n_pairs_w, n_chunks, E)
    pb = _pe_table(S, D) + b[None, :]
    wlo = jnp.concatenate([W.T, jnp.zeros((E, D), jnp.float32)], axis=0)
    whi = jnp.concatenate([jnp.zeros((E, D), jnp.float32), W.T], axis=0)
    gm = gamma.reshape(1, D)
    bt = beta.reshape(1, D)

    BB = 16
    Bg = B // _G
    grid = (Bg // BB,)
    out_shape = jax.ShapeDtypeStruct((B, S, D), jnp.float32)

    embs = [sc_gather(seq_flat[g], table_ext) for g in range(_G)]

    out = None
    for g in range(_G):
        base = g * (Bg // BB)

        def shifted(b0, i):
            return (b0 + i, 0, 0)

        emb_spec = pl.BlockSpec((BB * HS, 2 * E), lambda i: (i, 0))
        common_specs = [
            pl.BlockSpec((2 * E, D), lambda i: (0, 0)),
            pl.BlockSpec((2 * E, D), lambda i: (0, 0)),
            pl.BlockSpec((S, D), lambda i: (0, 0)),
            pl.BlockSpec((1, D), lambda i: (0, 0)),
            pl.BlockSpec((1, D), lambda i: (0, 0)),
        ]
        out_spec = pl.BlockSpec((BB, S, D), functools.partial(shifted, base))
        cp = pltpu.CompilerParams(dimension_semantics=("arbitrary",))
        if g == 0:
            out = pl.pallas_call(
                functools.partial(_tc_body, BB, HS),
                grid=grid,
                in_specs=[emb_spec] + common_specs,
                out_specs=out_spec,
                out_shape=out_shape,
                compiler_params=cp,
            )(embs[g], wlo, whi, pb, gm, bt)
        else:
            out = pl.pallas_call(
                functools.partial(_tc_body_chunk, BB, HS),
                grid=grid,
                in_specs=[pl.BlockSpec(memory_space=pl.ANY),
                          emb_spec] + common_specs,
                out_specs=out_spec,
                out_shape=out_shape,
                input_output_aliases={0: 0},
                compiler_params=cp,
            )(out, embs[g], wlo, whi, pb, gm, bt)
    return out


# speedup vs baseline: 1.2448x; 1.0001x over previous
import functools
import numpy as np
import jax
import jax.numpy as jnp
from jax import lax
from jax.experimental import pallas as pl
from jax.experimental.pallas import tpu as pltpu
from jax.experimental.pallas import tpu_sc as plsc

_EPS = 1e-5
_MAXLEN = 512

_NW = 32
_C = 640
_KBUF = 2
_G = 4


def _pe_table(S, D):
    pos = np.arange(_MAXLEN, dtype=np.float32)[:, None]
    i = np.arange(0, D, 2, dtype=np.float32)
    div = np.exp(-(np.log(10000.0)) * i / D)
    pe = np.zeros((_MAXLEN, D), dtype=np.float32)
    pe[:, 0::2] = np.sin(pos * div)
    pe[:, 1::2] = np.cos(pos * div)
    return jnp.asarray(pe[:S])


def _make_sc_gather(n_pairs_w, n_chunks, E):
    mesh = plsc.VectorSubcoreMesh(core_axis_name="c", subcore_axis_name="s")
    idx_scratch = [pltpu.VMEM((_C,), jnp.int32) for _ in range(_KBUF)]
    row_scratch = [pltpu.VMEM((_C, E), jnp.float32) for _ in range(_KBUF)]
    HC = _C // 2

    @functools.partial(
        pl.kernel,
        mesh=mesh,
        out_type=jax.ShapeDtypeStruct((_NW * n_pairs_w, 2 * E), jnp.float32),
        scratch_types=idx_scratch + row_scratch + [
            pltpu.SemaphoreType.DMA,
        ],
        compiler_params=pltpu.CompilerParams(use_tc_tiling_on_sc=False),
    )
    def sc_gather(seq_hbm, table_hbm, out_hbm, *scratch):
        idx_v = scratch[:_KBUF]
        rows_v = scratch[_KBUF : 2 * _KBUF]
        sem = scratch[2 * _KBUF]
        wid = lax.axis_index("s") * 2 + lax.axis_index("c")

        def group(g, _):
            copies = []
            for bf in range(_KBUF):
                pltpu.sync_copy(seq_hbm.at[wid, g * _KBUF + bf], idx_v[bf])
                cp = pltpu.async_copy(
                    table_hbm.at[idx_v[bf]], rows_v[bf], sem
                )
                copies.append(cp)
            for bf in range(_KBUF):
                ch = g * _KBUF + bf
                pairbase = wid * n_pairs_w + ch * HC
                copies[bf].wait()
                pltpu.sync_copy(
                    rows_v[bf].at[pl.ds(0, HC)],
                    out_hbm.at[pl.ds(pairbase, HC), pl.ds(0, E)],
                )
                pltpu.sync_copy(
                    rows_v[bf].at[pl.ds(HC, HC)],
                    out_hbm.at[pl.ds(pairbase, HC), pl.ds(E, E)],
                )
            return _

        lax.fori_loop(0, n_chunks // _KBUF, group, None)

    return sc_gather


def _tc_body(BB, HS, emb_ref, wlo_ref, whi_ref, pb_ref, g_ref,
             bt_ref, out_ref):
    wlo = wlo_ref[...]
    whi = whi_ref[...]
    pb = pb_ref[...]
    gm = g_ref[...]
    bt = bt_ref[...]

    def ln(x):
        mu = jnp.mean(x, axis=-1, keepdims=True)
        xc = x - mu
        var = jnp.mean(xc * xc, axis=-1, keepdims=True)
        return (xc * lax.rsqrt(var + _EPS)) * gm + bt

    p_all = emb_ref[...]
    x_lo_all = jnp.dot(p_all, wlo, preferred_element_type=jnp.float32)
    x_hi_all = jnp.dot(p_all, whi, preferred_element_type=jnp.float32)
    for i in range(BB):
        x_lo = x_lo_all[i * HS : (i + 1) * HS]
        out_ref[i, 0:HS] = ln(x_lo + pb[0:HS])
        x_hi = x_hi_all[i * HS : (i + 1) * HS]
        out_ref[i, HS : 2 * HS] = ln(x_hi + pb[HS : 2 * HS])


def _tc_body_chunk(BB, HS, prev_unused, *rest):
    del prev_unused
    _tc_body(BB, HS, *rest)


def kernel(sequence, table, W, b, gamma, beta):
    B, S = sequence.shape
    V, E = table.shape
    D = W.shape[0]
    N = B * S
    HS = S // 2
    Ng = N // _G
    n_w = Ng // _NW
    n_pairs_w = n_w // 2
    assert S % 2 == 0 and n_w % (_KBUF * _C) == 0 and B % _G == 0
    n_chunks = n_w // _C

    table_ext = jnp.concatenate(
        [table, jnp.zeros((8, E), jnp.float32)], axis=0
    )
    seq_i32 = sequence.astype(jnp.int32)
    seq_remap = jnp.where(seq_i32 == 0, V, seq_i32)
    lo = seq_remap[:, :HS].reshape(_G, _NW, n_chunks, _C // 2)
    hi = seq_remap[:, HS:].reshape(_G, _NW, n_chunks, _C // 2)
    seq_flat = jnp.concatenate([lo, hi], axis=-1)

    sc_gather = _make_sc_gather(n_pairs_w, n_chunks, E)
    pb = _pe_table(S, D) + b[None, :]
    wlo = jnp.concatenate([W.T, jnp.zeros((E, D), jnp.float32)], axis=0)
    whi = jnp.concatenate([jnp.zeros((E, D), jnp.float32), W.T], axis=0)
    gm = gamma.reshape(1, D)
    bt = beta.reshape(1, D)

    BB = 16
    Bg = B // _G
    grid = (Bg // BB,)
    out_shape = jax.ShapeDtypeStruct((B, S, D), jnp.float32)

    embs = [sc_gather(seq_flat[g], table_ext) for g in range(_G)]

    out = None
    for g in range(_G):
        base = g * (Bg // BB)

        def shifted(b0, i):
            return (b0 + i, 0, 0)

        emb_spec = pl.BlockSpec((BB * HS, 2 * E), lambda i: (i, 0))
        common_specs = [
            pl.BlockSpec((2 * E, D), lambda i: (0, 0)),
            pl.BlockSpec((2 * E, D), lambda i: (0, 0)),
            pl.BlockSpec((S, D), lambda i: (0, 0)),
            pl.BlockSpec((1, D), lambda i: (0, 0)),
            pl.BlockSpec((1, D), lambda i: (0, 0)),
        ]
        out_spec = pl.BlockSpec((BB, S, D), functools.partial(shifted, base))
        cp = pltpu.CompilerParams(dimension_semantics=("arbitrary",))
        if g == 0:
            out = pl.pallas_call(
                functools.partial(_tc_body, BB, HS),
                grid=grid,
                in_specs=[emb_spec] + common_specs,
                out_specs=out_spec,
                out_shape=out_shape,
                compiler_params=cp,
            )(embs[g], wlo, whi, pb, gm, bt)
        else:
            out = pl.pallas_call(
                functools.partial(_tc_body_chunk, BB, HS),
                grid=grid,
                in_specs=[pl.BlockSpec(memory_space=pl.ANY),
                          emb_spec] + common_specs,
                out_specs=out_spec,
                out_shape=out_shape,
                input_output_aliases={0: 0},
                compiler_params=cp,
            )(out, embs[g], wlo, whi, pb, gm, bt)
    return out


# speedup vs baseline: 1.3768x; 1.1061x over previous
import functools
import numpy as np
import jax
import jax.numpy as jnp
from jax import lax
from jax.experimental import pallas as pl
from jax.experimental.pallas import tpu as pltpu
from jax.experimental.pallas import tpu_sc as plsc

_EPS = 1e-5
_MAXLEN = 512

_NW = 32
_C = 400
_KBUF = 2
_G = 4


def _pe_table(S, D):
    pos = np.arange(_MAXLEN, dtype=np.float32)[:, None]
    i = np.arange(0, D, 2, dtype=np.float32)
    div = np.exp(-(np.log(10000.0)) * i / D)
    pe = np.zeros((_MAXLEN, D), dtype=np.float32)
    pe[:, 0::2] = np.sin(pos * div)
    pe[:, 1::2] = np.cos(pos * div)
    return jnp.asarray(pe[:S])


def _prep_body(RB, last, table_ref, out_ref):
    pid = pl.program_id(0)
    f = jnp.where(pid <= last, 1.0, 0.0).astype(jnp.float32)
    E = table_ref.shape[1]
    out_ref[:, 0:E] = table_ref[...] * f
    out_ref[:, E : 2 * E] = jnp.zeros((RB, E), jnp.float32)


def _prep_table(table):
    V, E = table.shape
    RB = 8000
    nblk = V // RB
    assert V % RB == 0
    Vext = (nblk + 1) * RB
    return pl.pallas_call(
        functools.partial(_prep_body, RB, nblk - 1),
        grid=(nblk + 1,),
        in_specs=[
            pl.BlockSpec((RB, E), lambda i, n=nblk - 1: (jnp.minimum(i, n), 0))
        ],
        out_specs=pl.BlockSpec((RB, 2 * E), lambda i: (i, 0)),
        out_shape=jax.ShapeDtypeStruct((Vext, 2 * E), jnp.float32),
        compiler_params=pltpu.CompilerParams(
            dimension_semantics=("arbitrary",),
        ),
    )(table)


def _make_sc_gather(n_pairs_w, n_chunks, E, Vext):
    mesh = plsc.VectorSubcoreMesh(core_axis_name="c", subcore_axis_name="s")
    idx_scratch = [pltpu.VMEM((_C,), jnp.int32) for _ in range(_KBUF)]
    row_scratch = [
        pltpu.VMEM((_C, 2 * E), jnp.float32) for _ in range(_KBUF)
    ]
    HC = _C // 2

    @functools.partial(
        pl.kernel,
        mesh=mesh,
        out_type=jax.ShapeDtypeStruct((_NW * n_pairs_w, 2 * E), jnp.float32),
        scratch_types=idx_scratch + row_scratch + [
            pltpu.SemaphoreType.DMA,
        ],
        compiler_params=pltpu.CompilerParams(use_tc_tiling_on_sc=False),
    )
    def sc_gather(seq_hbm, table_hbm, out_hbm, *scratch):
        idx_v = scratch[:_KBUF]
        rows_v = scratch[_KBUF : 2 * _KBUF]
        sem = scratch[2 * _KBUF]
        wid = lax.axis_index("s") * 2 + lax.axis_index("c")

        def group(g, _):
            copies = []
            for bf in range(_KBUF):
                pltpu.sync_copy(seq_hbm.at[wid, g * _KBUF + bf], idx_v[bf])
                cp = pltpu.async_copy(
                    table_hbm.at[idx_v[bf]], rows_v[bf], sem
                )
                copies.append(cp)
            for bf in range(_KBUF):
                ch = g * _KBUF + bf
                pairbase = wid * n_pairs_w + ch * HC
                copies[bf].wait()
                pltpu.sync_copy(
                    rows_v[bf].at[pl.ds(0, HC), pl.ds(0, E)],
                    out_hbm.at[pl.ds(pairbase, HC), pl.ds(0, E)],
                )
                pltpu.sync_copy(
                    rows_v[bf].at[pl.ds(HC, HC), pl.ds(0, E)],
                    out_hbm.at[pl.ds(pairbase, HC), pl.ds(E, E)],
                )
            return _

        lax.fori_loop(0, n_chunks // _KBUF, group, None)

    return sc_gather


def _tc_body(BB, HS, emb_ref, wlo_ref, whi_ref, pb_ref, g_ref,
             bt_ref, out_ref):
    wlo = wlo_ref[...]
    whi = whi_ref[...]
    pb = pb_ref[...]
    gm = g_ref[...]
    bt = bt_ref[...]

    def ln(x):
        mu = jnp.mean(x, axis=-1, keepdims=True)
        xc = x - mu
        var = jnp.mean(xc * xc, axis=-1, keepdims=True)
        return (xc * lax.rsqrt(var + _EPS)) * gm + bt

    p_all = emb_ref[...]
    x_lo_all = jnp.dot(p_all, wlo, preferred_element_type=jnp.float32)
    x_hi_all = jnp.dot(p_all, whi, preferred_element_type=jnp.float32)
    for i in range(BB):
        x_lo = x_lo_all[i * HS : (i + 1) * HS]
        out_ref[i, 0:HS] = ln(x_lo + pb[0:HS])
        x_hi = x_hi_all[i * HS : (i + 1) * HS]
        out_ref[i, HS : 2 * HS] = ln(x_hi + pb[HS : 2 * HS])


def _tc_body_chunk(BB, HS, prev_unused, *rest):
    del prev_unused
    _tc_body(BB, HS, *rest)


def kernel(sequence, table, W, b, gamma, beta):
    B, S = sequence.shape
    V, E = table.shape
    D = W.shape[0]
    N = B * S
    HS = S // 2
    Ng = N // _G
    n_w = Ng // _NW
    n_pairs_w = n_w // 2
    assert S % 2 == 0 and n_w % (_KBUF * _C) == 0 and B % _G == 0
    n_chunks = n_w // _C

    table_wide = _prep_table(table)
    Vext = table_wide.shape[0]
    seq_i32 = sequence.astype(jnp.int32)
    seq_remap = jnp.where(seq_i32 == 0, V, seq_i32)
    lo = seq_remap[:, :HS].reshape(_G, _NW, n_chunks, _C // 2)
    hi = seq_remap[:, HS:].reshape(_G, _NW, n_chunks, _C // 2)
    seq_flat = jnp.concatenate([lo, hi], axis=-1)

    sc_gather = _make_sc_gather(n_pairs_w, n_chunks, E, Vext)
    pb = _pe_table(S, D) + b[None, :]
    wlo = jnp.concatenate([W.T, jnp.zeros((E, D), jnp.float32)], axis=0)
    whi = jnp.concatenate([jnp.zeros((E, D), jnp.float32), W.T], axis=0)
    gm = gamma.reshape(1, D)
    bt = beta.reshape(1, D)

    BB = 16
    Bg = B // _G
    grid = (Bg // BB,)
    out_shape = jax.ShapeDtypeStruct((B, S, D), jnp.float32)

    embs = [sc_gather(seq_flat[g], table_wide) for g in range(_G)]

    out = None
    for g in range(_G):
        base = g * (Bg // BB)

        def shifted(b0, i):
            return (b0 + i, 0, 0)

        emb_spec = pl.BlockSpec((BB * HS, 2 * E), lambda i: (i, 0))
        common_specs = [
            pl.BlockSpec((2 * E, D), lambda i: (0, 0)),
            pl.BlockSpec((2 * E, D), lambda i: (0, 0)),
            pl.BlockSpec((S, D), lambda i: (0, 0)),
            pl.BlockSpec((1, D), lambda i: (0, 0)),
            pl.BlockSpec((1, D), lambda i: (0, 0)),
        ]
        out_spec = pl.BlockSpec((BB, S, D), functools.partial(shifted, base))
        cp = pltpu.CompilerParams(dimension_semantics=("arbitrary",))
        if g == 0:
            out = pl.pallas_call(
                functools.partial(_tc_body, BB, HS),
                grid=grid,
                in_specs=[emb_spec] + common_specs,
                out_specs=out_spec,
                out_shape=out_shape,
                compiler_params=cp,
            )(embs[g], wlo, whi, pb, gm, bt)
        else:
            out = pl.pallas_call(
                functools.partial(_tc_body_chunk, BB, HS),
                grid=grid,
                in_specs=[pl.BlockSpec(memory_space=pl.ANY),
                          emb_spec] + common_specs,
                out_specs=out_spec,
                out_shape=out_shape,
                input_output_aliases={0: 0},
                compiler_params=cp,
            )(out, embs[g], wlo, whi, pb, gm, bt)
    return out


# speedup vs baseline: 1.4366x; 1.0434x over previous
import functools
import numpy as np
import jax
import jax.numpy as jnp
from jax import lax
from jax.experimental import pallas as pl
from jax.experimental.pallas import tpu as pltpu
from jax.experimental.pallas import tpu_sc as plsc

_EPS = 1e-5
_MAXLEN = 512

_NW = 32
_C = 400
_KBUF = 2
_G = 4


def _pe_table(S, D):
    pos = np.arange(_MAXLEN, dtype=np.float32)[:, None]
    i = np.arange(0, D, 2, dtype=np.float32)
    div = np.exp(-(np.log(10000.0)) * i / D)
    pe = np.zeros((_MAXLEN, D), dtype=np.float32)
    pe[:, 0::2] = np.sin(pos * div)
    pe[:, 1::2] = np.cos(pos * div)
    return jnp.asarray(pe[:S])


def _prep_body(RB, last, table_ref, out_ref):
    pid = pl.program_id(0)
    f = jnp.where(pid <= last, 1.0, 0.0).astype(jnp.float32)
    E = table_ref.shape[1]
    out_ref[:, 0:E] = table_ref[...] * f
    out_ref[:, E : 2 * E] = jnp.zeros((RB, E), jnp.float32)


def _prep_table(table):
    V, E = table.shape
    RB = 8000
    nblk = V // RB
    assert V % RB == 0
    Vext = (nblk + 1) * RB
    return pl.pallas_call(
        functools.partial(_prep_body, RB, nblk - 1),
        grid=(nblk + 1,),
        in_specs=[
            pl.BlockSpec((RB, E), lambda i, n=nblk - 1: (jnp.minimum(i, n), 0))
        ],
        out_specs=pl.BlockSpec((RB, 2 * E), lambda i: (i, 0)),
        out_shape=jax.ShapeDtypeStruct((Vext, 2 * E), jnp.float32),
        compiler_params=pltpu.CompilerParams(
            dimension_semantics=("arbitrary",),
        ),
    )(table)


def _make_sc_gather(n_pairs_w, n_chunks, E, Vext):
    mesh = plsc.VectorSubcoreMesh(core_axis_name="c", subcore_axis_name="s")
    idx_scratch = [pltpu.VMEM((_C,), jnp.int32) for _ in range(_KBUF)]
    row_scratch = [pltpu.VMEM((_C, E), jnp.float32) for _ in range(_KBUF)]
    HC = _C // 2

    @functools.partial(
        pl.kernel,
        mesh=mesh,
        out_type=jax.ShapeDtypeStruct((_NW * n_pairs_w, 2 * E), jnp.float32),
        scratch_types=idx_scratch + row_scratch + [
            pltpu.SemaphoreType.DMA,
        ],
        compiler_params=pltpu.CompilerParams(use_tc_tiling_on_sc=False),
    )
    def sc_gather(seq_hbm, table_hbm, out_hbm, *scratch):
        idx_v = scratch[:_KBUF]
        rows_v = scratch[_KBUF : 2 * _KBUF]
        sem = scratch[2 * _KBUF]
        wid = lax.axis_index("s") * 2 + lax.axis_index("c")

        def group(g, _):
            copies = []
            for bf in range(_KBUF):
                pltpu.sync_copy(seq_hbm.at[wid, g * _KBUF + bf], idx_v[bf])
                cp = pltpu.async_copy(
                    table_hbm.at[idx_v[bf]], rows_v[bf], sem
                )
                copies.append(cp)
            for bf in range(_KBUF):
                ch = g * _KBUF + bf
                pairbase = wid * n_pairs_w + ch * HC
                copies[bf].wait()
                pltpu.sync_copy(
                    rows_v[bf].at[pl.ds(0, HC)],
                    out_hbm.at[pl.ds(pairbase, HC), pl.ds(0, E)],
                )
                pltpu.sync_copy(
                    rows_v[bf].at[pl.ds(HC, HC)],
                    out_hbm.at[pl.ds(pairbase, HC), pl.ds(E, E)],
                )
            return _

        lax.fori_loop(0, n_chunks // _KBUF, group, None)

    return sc_gather


def _tc_body(BB, HS, emb_ref, wlo_ref, whi_ref, pb_ref, g_ref,
             bt_ref, out_ref):
    wlo = wlo_ref[...]
    whi = whi_ref[...]
    pb = pb_ref[...]
    gm = g_ref[...]
    bt = bt_ref[...]

    def ln(x):
        mu = jnp.mean(x, axis=-1, keepdims=True)
        xc = x - mu
        var = jnp.mean(xc * xc, axis=-1, keepdims=True)
        return (xc * lax.rsqrt(var + _EPS)) * gm + bt

    p_all = emb_ref[...]
    x_lo_all = jnp.dot(p_all, wlo, preferred_element_type=jnp.float32)
    x_hi_all = jnp.dot(p_all, whi, preferred_element_type=jnp.float32)
    for i in range(BB):
        x_lo = x_lo_all[i * HS : (i + 1) * HS]
        out_ref[i, 0:HS] = ln(x_lo + pb[0:HS])
        x_hi = x_hi_all[i * HS : (i + 1) * HS]
        out_ref[i, HS : 2 * HS] = ln(x_hi + pb[HS : 2 * HS])


def _tc_body_chunk(BB, HS, prev_unused, *rest):
    del prev_unused
    _tc_body(BB, HS, *rest)


def kernel(sequence, table, W, b, gamma, beta):
    B, S = sequence.shape
    V, E = table.shape
    D = W.shape[0]
    N = B * S
    HS = S // 2
    Ng = N // _G
    n_w = Ng // _NW
    n_pairs_w = n_w // 2
    assert S % 2 == 0 and n_w % (_KBUF * _C) == 0 and B % _G == 0
    n_chunks = n_w // _C

    table_wide = _prep_table(table)
    Vext = table_wide.shape[0]
    table_rows = table_wide.reshape(2 * Vext, E)
    seq_i32 = sequence.astype(jnp.int32)
    seq_remap = jnp.where(seq_i32 == 0, 1, seq_i32 * 2)
    lo = seq_remap[:, :HS].reshape(_G, _NW, n_chunks, _C // 2)
    hi = seq_remap[:, HS:].reshape(_G, _NW, n_chunks, _C // 2)
    seq_flat = jnp.concatenate([lo, hi], axis=-1)

    sc_gather = _make_sc_gather(n_pairs_w, n_chunks, E, Vext)
    pb = _pe_table(S, D) + b[None, :]
    wlo = jnp.concatenate([W.T, jnp.zeros((E, D), jnp.float32)], axis=0)
    whi = jnp.concatenate([jnp.zeros((E, D), jnp.float32), W.T], axis=0)
    gm = gamma.reshape(1, D)
    bt = beta.reshape(1, D)

    BB = 16
    Bg = B // _G
    grid = (Bg // BB,)
    out_shape = jax.ShapeDtypeStruct((B, S, D), jnp.float32)

    embs = [sc_gather(seq_flat[g], table_rows) for g in range(_G)]

    out = None
    for g in range(_G):
        base = g * (Bg // BB)

        def shifted(b0, i):
            return (b0 + i, 0, 0)

        emb_spec = pl.BlockSpec((BB * HS, 2 * E), lambda i: (i, 0))
        common_specs = [
            pl.BlockSpec((2 * E, D), lambda i: (0, 0)),
            pl.BlockSpec((2 * E, D), lambda i: (0, 0)),
            pl.BlockSpec((S, D), lambda i: (0, 0)),
            pl.BlockSpec((1, D), lambda i: (0, 0)),
            pl.BlockSpec((1, D), lambda i: (0, 0)),
        ]
        out_spec = pl.BlockSpec((BB, S, D), functools.partial(shifted, base))
        cp = pltpu.CompilerParams(dimension_semantics=("arbitrary",))
        if g == 0:
            out = pl.pallas_call(
                functools.partial(_tc_body, BB, HS),
                grid=grid,
                in_specs=[emb_spec] + common_specs,
                out_specs=out_spec,
                out_shape=out_shape,
                compiler_params=cp,
            )(embs[g], wlo, whi, pb, gm, bt)
        else:
            out = pl.pallas_call(
                functools.partial(_tc_body_chunk, BB, HS),
                grid=grid,
                in_specs=[pl.BlockSpec(memory_space=pl.ANY),
                          emb_spec] + common_specs,
                out_specs=out_spec,
                out_shape=out_shape,
                input_output_aliases={0: 0},
                compiler_params=cp,
            )(out, embs[g], wlo, whi, pb, gm, bt)
    return out


# speedup vs baseline: 1.4402x; 1.0025x over previous
import functools
import numpy as np
import jax
import jax.numpy as jnp
from jax import lax
from jax.experimental import pallas as pl
from jax.experimental.pallas import tpu as pltpu
from jax.experimental.pallas import tpu_sc as plsc

_EPS = 1e-5
_MAXLEN = 512

_NW = 32
_C = 640
_KBUF = 2
_G = 4


def _pe_table(S, D):
    pos = np.arange(_MAXLEN, dtype=np.float32)[:, None]
    i = np.arange(0, D, 2, dtype=np.float32)
    div = np.exp(-(np.log(10000.0)) * i / D)
    pe = np.zeros((_MAXLEN, D), dtype=np.float32)
    pe[:, 0::2] = np.sin(pos * div)
    pe[:, 1::2] = np.cos(pos * div)
    return jnp.asarray(pe[:S])


def _prep_body(RB, last, table_ref, out_ref):
    pid = pl.program_id(0)
    f = jnp.where(pid <= last, 1.0, 0.0).astype(jnp.float32)
    E = table_ref.shape[1]
    out_ref[:, 0:E] = table_ref[...] * f
    out_ref[:, E : 2 * E] = jnp.zeros((RB, E), jnp.float32)


def _prep_table(table):
    V, E = table.shape
    RB = 8000
    nblk = V // RB
    assert V % RB == 0
    Vext = (nblk + 1) * RB
    return pl.pallas_call(
        functools.partial(_prep_body, RB, nblk - 1),
        grid=(nblk + 1,),
        in_specs=[
            pl.BlockSpec((RB, E), lambda i, n=nblk - 1: (jnp.minimum(i, n), 0))
        ],
        out_specs=pl.BlockSpec((RB, 2 * E), lambda i: (i, 0)),
        out_shape=jax.ShapeDtypeStruct((Vext, 2 * E), jnp.float32),
        compiler_params=pltpu.CompilerParams(
            dimension_semantics=("arbitrary",),
        ),
    )(table)


def _make_sc_gather(n_pairs_w, n_chunks, E, Vext):
    mesh = plsc.VectorSubcoreMesh(core_axis_name="c", subcore_axis_name="s")
    idx_scratch = [pltpu.VMEM((_C,), jnp.int32) for _ in range(_KBUF)]
    row_scratch = [pltpu.VMEM((_C, E), jnp.float32) for _ in range(_KBUF)]
    HC = _C // 2

    @functools.partial(
        pl.kernel,
        mesh=mesh,
        out_type=jax.ShapeDtypeStruct((_NW * n_pairs_w, 2 * E), jnp.float32),
        scratch_types=idx_scratch + row_scratch + [
            pltpu.SemaphoreType.DMA,
        ],
        compiler_params=pltpu.CompilerParams(use_tc_tiling_on_sc=False),
    )
    def sc_gather(seq_hbm, table_hbm, out_hbm, *scratch):
        idx_v = scratch[:_KBUF]
        rows_v = scratch[_KBUF : 2 * _KBUF]
        sem = scratch[2 * _KBUF]
        wid = lax.axis_index("s") * 2 + lax.axis_index("c")

        def group(g, _):
            copies = []
            for bf in range(_KBUF):
                pltpu.sync_copy(seq_hbm.at[wid, g * _KBUF + bf], idx_v[bf])
                cp = pltpu.async_copy(
                    table_hbm.at[idx_v[bf]], rows_v[bf], sem
                )
                copies.append(cp)
            for bf in range(_KBUF):
                ch = g * _KBUF + bf
                pairbase = wid * n_pairs_w + ch * HC
                copies[bf].wait()
                pltpu.sync_copy(
                    rows_v[bf].at[pl.ds(0, HC)],
                    out_hbm.at[pl.ds(pairbase, HC), pl.ds(0, E)],
                )
                pltpu.sync_copy(
                    rows_v[bf].at[pl.ds(HC, HC)],
                    out_hbm.at[pl.ds(pairbase, HC), pl.ds(E, E)],
                )
            return _

        lax.fori_loop(0, n_chunks // _KBUF, group, None)

    return sc_gather


def _tc_body(BB, HS, emb_ref, wlo_ref, whi_ref, pb_ref, g_ref,
             bt_ref, out_ref):
    wlo = wlo_ref[...]
    whi = whi_ref[...]
    pb = pb_ref[...]
    gm = g_ref[...]
    bt = bt_ref[...]

    def ln(x):
        mu = jnp.mean(x, axis=-1, keepdims=True)
        xc = x - mu
        var = jnp.mean(xc * xc, axis=-1, keepdims=True)
        return (xc * lax.rsqrt(var + _EPS)) * gm + bt

    p_all = emb_ref[...]
    x_lo_all = jnp.dot(p_all, wlo, preferred_element_type=jnp.float32)
    x_hi_all = jnp.dot(p_all, whi, preferred_element_type=jnp.float32)
    for i in range(BB):
        x_lo = x_lo_all[i * HS : (i + 1) * HS]
        out_ref[i, 0:HS] = ln(x_lo + pb[0:HS])
        x_hi = x_hi_all[i * HS : (i + 1) * HS]
        out_ref[i, HS : 2 * HS] = ln(x_hi + pb[HS : 2 * HS])


def _tc_body_chunk(BB, HS, prev_unused, *rest):
    del prev_unused
    _tc_body(BB, HS, *rest)


def kernel(sequence, table, W, b, gamma, beta):
    B, S = sequence.shape
    V, E = table.shape
    D = W.shape[0]
    N = B * S
    HS = S // 2
    Ng = N // _G
    n_w = Ng // _NW
    n_pairs_w = n_w // 2
    assert S % 2 == 0 and n_w % (_KBUF * _C) == 0 and B % _G == 0
    n_chunks = n_w // _C

    table_wide = _prep_table(table)
    Vext = table_wide.shape[0]
    table_rows = table_wide.reshape(2 * Vext, E)
    seq_i32 = sequence.astype(jnp.int32)
    seq_remap = jnp.where(seq_i32 == 0, 1, seq_i32 * 2)
    lo = seq_remap[:, :HS].reshape(_G, _NW, n_chunks, _C // 2)
    hi = seq_remap[:, HS:].reshape(_G, _NW, n_chunks, _C // 2)
    seq_flat = jnp.concatenate([lo, hi], axis=-1)

    sc_gather = _make_sc_gather(n_pairs_w, n_chunks, E, Vext)
    pb = _pe_table(S, D) + b[None, :]
    wlo = jnp.concatenate([W.T, jnp.zeros((E, D), jnp.float32)], axis=0)
    whi = jnp.concatenate([jnp.zeros((E, D), jnp.float32), W.T], axis=0)
    gm = gamma.reshape(1, D)
    bt = beta.reshape(1, D)

    BB = 16
    Bg = B // _G
    grid = (Bg // BB,)
    out_shape = jax.ShapeDtypeStruct((B, S, D), jnp.float32)

    embs = [sc_gather(seq_flat[g], table_rows) for g in range(_G)]

    out = None
    for g in range(_G):
        base = g * (Bg // BB)

        def shifted(b0, i):
            return (b0 + i, 0, 0)

        emb_spec = pl.BlockSpec((BB * HS, 2 * E), lambda i: (i, 0))
        common_specs = [
            pl.BlockSpec((2 * E, D), lambda i: (0, 0)),
            pl.BlockSpec((2 * E, D), lambda i: (0, 0)),
            pl.BlockSpec((S, D), lambda i: (0, 0)),
            pl.BlockSpec((1, D), lambda i: (0, 0)),
            pl.BlockSpec((1, D), lambda i: (0, 0)),
        ]
        out_spec = pl.BlockSpec((BB, S, D), functools.partial(shifted, base))
        cp = pltpu.CompilerParams(dimension_semantics=("arbitrary",))
        if g == 0:
            out = pl.pallas_call(
                functools.partial(_tc_body, BB, HS),
                grid=grid,
                in_specs=[emb_spec] + common_specs,
                out_specs=out_spec,
                out_shape=out_shape,
                compiler_params=cp,
            )(embs[g], wlo, whi, pb, gm, bt)
        else:
            out = pl.pallas_call(
                functools.partial(_tc_body_chunk, BB, HS),
                grid=grid,
                in_specs=[pl.BlockSpec(memory_space=pl.ANY),
                          emb_spec] + common_specs,
                out_specs=out_spec,
                out_shape=out_shape,
                input_output_aliases={0: 0},
                compiler_params=cp,
            )(out, embs[g], wlo, whi, pb, gm, bt)
    return out
